# Initial kernel scaffold; baseline (speedup 1.0000x reference)
import jax
import jax.numpy as jnp
from jax.experimental import pallas as pl


def kernel(x, batch, edge, attn_W, attn_b):
    raise NotImplementedError("write your pallas kernel here")


# speedup vs baseline: 14.6932x; 14.6932x over previous
import functools

import jax
import jax.numpy as jnp
from jax.experimental import pallas as pl
from jax.experimental.pallas import tpu as pltpu

H = 4
DK = 32
HID = H * DK
B = 64
R = 128


def _fused_kernel(nblk, mpad,
                  base_ref, trip_ref,
                  xp_ref, bidx_ref, wbd_ref, sel_ref, selt_ref, bias_ref,
                  out_ref, y_ref):
    xp = xp_ref[...]
    t = jnp.dot(xp, wbd_ref[...], preferred_element_type=jnp.float32)
    p = t * xp
    s4 = jnp.dot(p, sel_ref[...], preferred_element_type=jnp.float32)
    s4 = s4 + bias_ref[...]

    gm = jnp.max(s4, axis=0, keepdims=True)
    e = jnp.exp(s4 - gm)
    bidx = bidx_ref[...]
    iota = jax.lax.broadcasted_iota(jnp.int32, (mpad, B), 1)
    oh = (bidx == iota).astype(jnp.float32)
    denom = jax.lax.dot_general(oh, e, (((0,), (0,)), ((), ())),
                                preferred_element_type=jnp.float32)
    dpm = jnp.dot(oh, denom, preferred_element_type=jnp.float32)
    a4 = e / (dpm + 1e-16)
    a128 = jnp.dot(a4, selt_ref[...], preferred_element_type=jnp.float32)
    y_ref[...] = a128 * xp

    def seg_body(b, carry):
        base = base_ref[b]
        trip = trip_ref[b]

        def chunk_body(j, acc):
            k = base + j
            yblk = y_ref[pl.ds(k * R, R), :]
            xblk = xp_ref[pl.ds(k * R, R), :]
            bblk = bidx_ref[pl.ds(k * R, R), :]
            msk = (bblk == b).astype(jnp.float32)
            ym = yblk * msk
            return acc + jax.lax.dot_general(
                ym, xblk, (((0,), (0,)), ((), ())),
                preferred_element_type=jnp.float32)

        acc0 = jnp.zeros((HID, HID), jnp.float32)
        out_ref[pl.ds(b, 1)] = jax.lax.fori_loop(0, trip, chunk_body, acc0)[None]
        return carry

    jax.lax.fori_loop(0, B, seg_body, 0)


def kernel(x, batch, edge, attn_W, attn_b):
    del edge
    m = x.shape[0]
    nblk = (m + R - 1) // R
    mpad = nblk * R

    xp = jnp.pad(x.astype(jnp.float32), ((0, mpad - m), (0, 0)))
    bi = batch.astype(jnp.int32)
    bip = jnp.pad(bi, (0, mpad - m), constant_values=B)[:, None]

    w3 = attn_W.reshape(H, DK, DK).astype(jnp.float32)
    wbd = jax.scipy.linalg.block_diag(*[w3[h] for h in range(H)])
    sel = jnp.repeat(jnp.eye(H, dtype=jnp.float32), DK, axis=0)
    selt = sel.T
    bias_row = attn_b.astype(jnp.float32)[None, :]

    segs = jnp.arange(B)
    starts = jnp.searchsorted(bi, segs, side="left")
    ends = jnp.searchsorted(bi, segs, side="right")
    base = (starts // R).astype(jnp.int32)
    trip = jnp.where(ends > starts,
                     (ends - 1) // R - starts // R + 1, 0).astype(jnp.int32)

    acc = pl.pallas_call(
        functools.partial(_fused_kernel, nblk, mpad),
        in_specs=[
            pl.BlockSpec(memory_space=pltpu.SMEM),
            pl.BlockSpec(memory_space=pltpu.SMEM),
            pl.BlockSpec(memory_space=pltpu.VMEM),
            pl.BlockSpec(memory_space=pltpu.VMEM),
            pl.BlockSpec(memory_space=pltpu.VMEM),
            pl.BlockSpec(memory_space=pltpu.VMEM),
            pl.BlockSpec(memory_space=pltpu.VMEM),
            pl.BlockSpec(memory_space=pltpu.VMEM),
        ],
        out_shape=jax.ShapeDtypeStruct((B, HID, HID), jnp.float32),
        scratch_shapes=[pltpu.VMEM((mpad, HID), jnp.float32)],
    )(base, trip, xp, bip, wbd, sel, selt, bias_row)

    out3 = jnp.stack([acc[:, h * DK:(h + 1) * DK, h * DK:(h + 1) * DK]
                      .reshape(B, DK * DK) for h in range(H)])
    return out3.reshape(B, H * DK * DK)


# speedup vs baseline: 17.3589x; 1.1814x over previous
import functools

import jax
import jax.numpy as jnp
from jax.experimental import pallas as pl
from jax.experimental.pallas import tpu as pltpu

H = 4
DK = 32
HID = H * DK
B = 64
R = 512


def _fused_kernel(nblk, mpad,
                  base_ref, trip_ref,
                  xp_ref, bidx_ref, wbd_ref, sel_ref, selt_ref, bias_ref,
                  out_ref, y_ref):
    xp = xp_ref[...]
    t = jnp.dot(xp, wbd_ref[...], preferred_element_type=jnp.float32)
    p = t * xp
    s4 = jnp.dot(p, sel_ref[...], preferred_element_type=jnp.float32)
    s4 = s4 + bias_ref[...]

    gm = jnp.max(s4, axis=0, keepdims=True)
    e = jnp.exp(s4 - gm)
    bidx = bidx_ref[...]
    iota = jax.lax.broadcasted_iota(jnp.int32, (mpad, B), 1)
    oh = (bidx == iota).astype(jnp.float32)
    denom = jax.lax.dot_general(oh, e, (((0,), (0,)), ((), ())),
                                preferred_element_type=jnp.float32)
    dpm = jnp.dot(oh, denom, preferred_element_type=jnp.float32)
    a4 = e / (dpm + 1e-16)
    a128 = jnp.dot(a4, selt_ref[...], preferred_element_type=jnp.float32)
    y_ref[...] = a128 * xp

    def seg_body(b, carry):
        base = base_ref[b]
        trip = trip_ref[b]

        def chunk_body(j, acc):
            k = base + j
            yblk = y_ref[pl.ds(k * R, R), :]
            xblk = xp_ref[pl.ds(k * R, R), :]
            bblk = bidx_ref[pl.ds(k * R, R), :]
            msk = (bblk == b).astype(jnp.float32)
            ym = yblk * msk
            return acc + jax.lax.dot_general(
                ym, xblk, (((0,), (0,)), ((), ())),
                preferred_element_type=jnp.float32)

        acc0 = jnp.zeros((HID, HID), jnp.float32)
        out_ref[pl.ds(b, 1)] = jax.lax.fori_loop(0, trip, chunk_body, acc0)[None]
        return carry

    jax.lax.fori_loop(0, B, seg_body, 0)


def kernel(x, batch, edge, attn_W, attn_b):
    del edge
    m = x.shape[0]
    nblk = (m + R - 1) // R
    mpad = nblk * R

    xp = jnp.pad(x.astype(jnp.float32), ((0, mpad - m), (0, 0)))
    bi = batch.astype(jnp.int32)
    bip = jnp.pad(bi, (0, mpad - m), constant_values=B)[:, None]

    w3 = attn_W.reshape(H, DK, DK).astype(jnp.float32)
    wbd = jax.scipy.linalg.block_diag(*[w3[h] for h in range(H)])
    sel = jnp.repeat(jnp.eye(H, dtype=jnp.float32), DK, axis=0)
    selt = sel.T
    bias_row = attn_b.astype(jnp.float32)[None, :]

    segs = jnp.arange(B)
    starts = jnp.searchsorted(bi, segs, side="left")
    ends = jnp.searchsorted(bi, segs, side="right")
    base = (starts // R).astype(jnp.int32)
    trip = jnp.where(ends > starts,
                     (ends - 1) // R - starts // R + 1, 0).astype(jnp.int32)

    acc = pl.pallas_call(
        functools.partial(_fused_kernel, nblk, mpad),
        in_specs=[
            pl.BlockSpec(memory_space=pltpu.SMEM),
            pl.BlockSpec(memory_space=pltpu.SMEM),
            pl.BlockSpec(memory_space=pltpu.VMEM),
            pl.BlockSpec(memory_space=pltpu.VMEM),
            pl.BlockSpec(memory_space=pltpu.VMEM),
            pl.BlockSpec(memory_space=pltpu.VMEM),
            pl.BlockSpec(memory_space=pltpu.VMEM),
            pl.BlockSpec(memory_space=pltpu.VMEM),
        ],
        out_shape=jax.ShapeDtypeStruct((B, HID, HID), jnp.float32),
        scratch_shapes=[pltpu.VMEM((mpad, HID), jnp.float32)],
    )(base, trip, xp, bip, wbd, sel, selt, bias_row)

    out3 = jnp.stack([acc[:, h * DK:(h + 1) * DK, h * DK:(h + 1) * DK]
                      .reshape(B, DK * DK) for h in range(H)])
    return out3.reshape(B, H * DK * DK)


# speedup vs baseline: 29.1896x; 1.6815x over previous
import functools

import jax
import jax.numpy as jnp
from jax.experimental import pallas as pl
from jax.experimental.pallas import tpu as pltpu

H = 4
DK = 32
HID = H * DK
B = 64
R = 512


def _fused_kernel(m, mpad,
                  x_ref, bt_ref, wbdt_ref,
                  base_ref, trip_ref,
                  out_ref, xp_ref, xt_ref, yt_ref, et_ref, bti_ref):
    xp_ref[0:m, :] = x_ref[...]
    xp_ref[m:mpad, :] = jnp.zeros((mpad - m, HID), jnp.float32)
    xt_ref[:, 0:m] = x_ref[...].T
    xt_ref[:, m:mpad] = jnp.zeros((HID, mpad - m), jnp.float32)
    bti_ref[:, 0:m] = bt_ref[...]
    bti_ref[:, m:mpad] = jnp.full((1, mpad - m), B, jnp.int32)

    xt = xt_ref[...]
    tt = jnp.dot(wbdt_ref[...], xt, preferred_element_type=jnp.float32)
    pt = tt * xt
    s4 = jnp.concatenate(
        [jnp.sum(pt[h * DK:(h + 1) * DK, :], axis=0, keepdims=True)
         for h in range(H)], axis=0)
    gm = jnp.max(s4, axis=1, keepdims=True)
    et = jnp.exp(s4 - gm)
    et_ref[...] = et
    e128 = jnp.concatenate(
        [jnp.broadcast_to(et[h:h + 1, :], (DK, mpad)) for h in range(H)],
        axis=0)
    yt_ref[...] = e128 * xt

    def seg_body(b, carry):
        base = base_ref[b]
        trip = trip_ref[b]

        def chunk_body(j, c):
            gram, dsum = c
            k = pl.multiple_of((base + j) * R, 128)
            msk = (bti_ref[:, pl.ds(k, R)] == b).astype(jnp.float32)
            ym = yt_ref[:, pl.ds(k, R)] * msk
            gram = gram + jnp.dot(ym, xp_ref[pl.ds(k, R), :],
                                  preferred_element_type=jnp.float32)
            dsum = dsum + jnp.sum(et_ref[:, pl.ds(k, R)] * msk,
                                  axis=1, keepdims=True)
            return gram, dsum

        gram, dsum = jax.lax.fori_loop(
            0, trip, chunk_body,
            (jnp.zeros((HID, HID), jnp.float32), jnp.zeros((H, 1), jnp.float32)))
        rec = 1.0 / (dsum + 1e-16)
        for h in range(H):
            blk = gram[h * DK:(h + 1) * DK, h * DK:(h + 1) * DK]
            out_ref[h, pl.ds(b, 1)] = (blk * rec[h:h + 1, 0:1])[None]
        return carry

    jax.lax.fori_loop(0, B, seg_body, 0)


def kernel(x, batch, edge, attn_W, attn_b):
    del edge, attn_b
    m = x.shape[0]
    mpad = ((m + R - 1) // R) * R

    xf = x.astype(jnp.float32)
    bi = batch.astype(jnp.int32)
    bt = bi[None, :]

    w3t = jnp.swapaxes(attn_W.reshape(H, DK, DK).astype(jnp.float32), 1, 2)
    wbdt = jax.scipy.linalg.block_diag(*[w3t[h] for h in range(H)])

    segs = jnp.arange(B)
    starts = jnp.searchsorted(bi, segs, side="left")
    ends = jnp.concatenate([starts[1:], jnp.array([m])])
    base = (starts // R).astype(jnp.int32)
    trip = jnp.where(ends > starts,
                     (ends - 1) // R - starts // R + 1, 0).astype(jnp.int32)

    out4 = pl.pallas_call(
        functools.partial(_fused_kernel, m, mpad),
        in_specs=[
            pl.BlockSpec(memory_space=pltpu.VMEM),
            pl.BlockSpec(memory_space=pltpu.VMEM),
            pl.BlockSpec(memory_space=pltpu.VMEM),
            pl.BlockSpec(memory_space=pltpu.SMEM),
            pl.BlockSpec(memory_space=pltpu.SMEM),
        ],
        out_shape=jax.ShapeDtypeStruct((H, B, DK, DK), jnp.float32),
        scratch_shapes=[pltpu.VMEM((mpad, HID), jnp.float32),
                        pltpu.VMEM((HID, mpad), jnp.float32),
                        pltpu.VMEM((HID, mpad), jnp.float32),
                        pltpu.VMEM((H, mpad), jnp.float32),
                        pltpu.VMEM((1, mpad), jnp.int32)],
    )(xf, bt, wbdt, base, trip)

    return out4.reshape(B, H * DK * DK)
